# baseline (speedup 1.0000x reference)
import functools

import jax
import jax.numpy as jnp
from jax import lax
from jax.experimental import pallas as pl
from jax.experimental.pallas import tpu as pltpu
from jax.experimental.pallas import tpu_sc as plsc

_N = 1024
_NC, _NS, _L = 2, 16, 16
_NW = _NC * _NS
_ROWS_PER_W = _N // _NW
_GROUPS = _ROWS_PER_W // _L
_UNROLL = 4
_BR = 256


def _vals_kernel(emb_full_ref, lab_row_ref, lab_col_ref,
                 pos_ref, neg_ref, en_ref):
    n = emb_full_ref.shape[0]
    i = pl.program_id(0)

    @pl.when(i == 0)
    def _():
        emb = emb_full_ref[...]
        sq = jnp.sum(emb * emb, axis=1, keepdims=True)
        en_ref[...] = emb * jax.lax.rsqrt(jnp.maximum(sq, 1e-30))

    en_full = en_ref[...]
    en_rows = en_ref[pl.ds(i * _BR, _BR), :]
    g = jax.lax.dot_general(en_rows, en_full, (((1,), (1,)), ((), ())),
                            preferred_element_type=jnp.float32,
                            precision=jax.lax.Precision.HIGHEST)
    dist = 1.0 - g

    lab_r = lab_row_ref[...]
    lab_c = lab_col_ref[...]
    row_ids = i * _BR + jax.lax.broadcasted_iota(jnp.int32, (_BR, n), 0)
    col_ids = jax.lax.broadcasted_iota(jnp.int32, (_BR, n), 1)
    same = lab_r == lab_c
    inf = jnp.float32(jnp.inf)
    pos_ref[...] = jnp.where(same & (row_ids != col_ids), dist, -inf)
    neg_ref[...] = jnp.where(same, inf, dist)


def _sc_reduce_body(pos_hbm, neg_hbm, pos_d_hbm, neg_d_hbm,
                    pos_i_hbm, neg_i_hbm,
                    pbuf0, nbuf0, pbuf1, nbuf1, spd, snd, spi, sni,
                    sem0, sem1):
    c = lax.axis_index("c")
    s = lax.axis_index("s")
    wid = s * _NC + c
    base = wid * _ROWS_PER_W
    inf = jnp.float32(jnp.inf)
    lane = lax.broadcasted_iota(jnp.int32, (_L,), 0)

    copies = ([], [])
    for t, (pbuf, nbuf, sem) in enumerate(((pbuf0, nbuf0, sem0),
                                           (pbuf1, nbuf1, sem1))):
        gbase = base + t * _L
        for l in range(_L):
            copies[t].append(pltpu.async_copy(
                pos_hbm.at[gbase + l], pbuf.at[l, pl.ds(0, _N)], sem))
            copies[t].append(pltpu.async_copy(
                neg_hbm.at[gbase + l], nbuf.at[l, pl.ds(0, _N)], sem))

    for t, (pbuf, nbuf) in enumerate(((pbuf0, nbuf0), (pbuf1, nbuf1))):
        for cp in copies[t]:
            cp.wait()

        def step(jj, accs):
            out = []
            for u in range(_UNROLL):
                bpv, bpi, bnv, bni = accs[u]
                j = jj * _UNROLL + u
                jf = j.astype(jnp.float32)
                jv = jnp.full((_L,), j, jnp.int32)
                pv = plsc.load_gather(pbuf, [lane, jv])
                nv = plsc.load_gather(nbuf, [lane, jv])
                bpi = jnp.where(pv > bpv, jf, bpi)
                bpv = jnp.maximum(bpv, pv)
                bni = jnp.where(nv < bnv, jf, bni)
                bnv = jnp.minimum(bnv, nv)
                out.append((bpv, bpi, bnv, bni))
            return tuple(out)

        init1 = (jnp.full((_L,), -inf), jnp.zeros((_L,), jnp.float32),
                 jnp.full((_L,), inf), jnp.zeros((_L,), jnp.float32))
        accs = lax.fori_loop(0, _N // _UNROLL, step, (init1,) * _UNROLL)

        big = jnp.full((_L,), jnp.float32(_N))
        pm = accs[0][0]
        nm = accs[0][2]
        for u in range(1, _UNROLL):
            pm = jnp.maximum(pm, accs[u][0])
            nm = jnp.minimum(nm, accs[u][2])
        pi = big
        ni = big
        for u in range(_UNROLL):
            pi = jnp.minimum(pi, jnp.where(accs[u][0] == pm, accs[u][1], big))
            ni = jnp.minimum(ni, jnp.where(accs[u][2] == nm, accs[u][3], big))
        pi = jnp.where(pm == -inf, 0.0, pi)
        ni = jnp.where(nm == inf, 0.0, ni)

        spd[pl.ds(t * _L, _L)] = pm
        spi[pl.ds(t * _L, _L)] = pi
        snd[pl.ds(t * _L, _L)] = nm
        sni[pl.ds(t * _L, _L)] = ni

    pltpu.sync_copy(spd, pos_d_hbm.at[pl.ds(base, _ROWS_PER_W)])
    pltpu.sync_copy(snd, neg_d_hbm.at[pl.ds(base, _ROWS_PER_W)])
    pltpu.sync_copy(spi, pos_i_hbm.at[pl.ds(base, _ROWS_PER_W)])
    pltpu.sync_copy(sni, neg_i_hbm.at[pl.ds(base, _ROWS_PER_W)])


def kernel(embeddings, labels):
    n, d = embeddings.shape
    lab_col = labels.reshape(1, n)
    lab_row = labels.reshape(n, 1)

    pos_vals, neg_vals = pl.pallas_call(
        _vals_kernel,
        grid=(n // _BR,),
        in_specs=[
            pl.BlockSpec((n, d), lambda i: (0, 0)),
            pl.BlockSpec((_BR, 1), lambda i: (i, 0)),
            pl.BlockSpec((1, n), lambda i: (0, 0)),
        ],
        out_specs=(
            pl.BlockSpec((_BR, n), lambda i: (i, 0)),
            pl.BlockSpec((_BR, n), lambda i: (i, 0)),
        ),
        out_shape=(
            jax.ShapeDtypeStruct((n, n), jnp.float32),
            jax.ShapeDtypeStruct((n, n), jnp.float32),
        ),
        scratch_shapes=[pltpu.VMEM((n, d), jnp.float32)],
    )(embeddings, lab_row, lab_col)

    sc_reduce = functools.partial(
        pl.kernel,
        out_type=(
            jax.ShapeDtypeStruct((n,), jnp.float32),
            jax.ShapeDtypeStruct((n,), jnp.float32),
            jax.ShapeDtypeStruct((n,), jnp.float32),
            jax.ShapeDtypeStruct((n,), jnp.float32),
        ),
        mesh=plsc.VectorSubcoreMesh(core_axis_name="c", subcore_axis_name="s"),
        scratch_types=[
            pltpu.VMEM((_L, _N + 1), jnp.float32),
            pltpu.VMEM((_L, _N + 1), jnp.float32),
            pltpu.VMEM((_L, _N + 1), jnp.float32),
            pltpu.VMEM((_L, _N + 1), jnp.float32),
            pltpu.VMEM((_ROWS_PER_W,), jnp.float32),
            pltpu.VMEM((_ROWS_PER_W,), jnp.float32),
            pltpu.VMEM((_ROWS_PER_W,), jnp.float32),
            pltpu.VMEM((_ROWS_PER_W,), jnp.float32),
            pltpu.SemaphoreType.DMA,
            pltpu.SemaphoreType.DMA,
        ],
        compiler_params=pltpu.CompilerParams(needs_layout_passes=False),
    )(_sc_reduce_body)

    pos_d, neg_d, pos_if, neg_if = sc_reduce(pos_vals, neg_vals)

    anchors = jnp.arange(n, dtype=jnp.int32)
    triplets = jnp.column_stack((anchors, pos_if.astype(jnp.int32),
                                 neg_if.astype(jnp.int32)))
    return (triplets, pos_d, neg_d)


# speedup vs baseline: 1.3569x; 1.3569x over previous
import functools

import jax
import jax.numpy as jnp
from jax import lax
from jax.experimental import pallas as pl
from jax.experimental.pallas import tpu as pltpu
from jax.experimental.pallas import tpu_sc as plsc

_N = 1024
_NC, _NS, _L = 2, 16, 16
_NW = _NC * _NS
_ROWS_PER_W = _N // _NW
_GROUPS = _ROWS_PER_W // _L
_UNROLL = 4
_CHUNKS = _N // _L
_BR = 256


def _vals_kernel(emb_full_ref, lab_row_ref, lab_col_ref,
                 pos_ref, neg_ref, en_ref):
    n = emb_full_ref.shape[0]
    i = pl.program_id(0)

    @pl.when(i == 0)
    def _():
        emb = emb_full_ref[...]
        sq = jnp.sum(emb * emb, axis=1, keepdims=True)
        en_ref[...] = emb * jax.lax.rsqrt(jnp.maximum(sq, 1e-30))

    en_full = en_ref[...]
    en_rows = en_ref[pl.ds(i * _BR, _BR), :]
    g = jax.lax.dot_general(en_rows, en_full, (((1,), (1,)), ((), ())),
                            preferred_element_type=jnp.float32,
                            precision=jax.lax.Precision.HIGHEST)
    dist = 1.0 - g

    lab_r = lab_row_ref[...]
    lab_c = lab_col_ref[...]
    row_ids = i * _BR + jax.lax.broadcasted_iota(jnp.int32, (_BR, n), 0)
    col_ids = jax.lax.broadcasted_iota(jnp.int32, (_BR, n), 1)
    same = lab_r == lab_c
    inf = jnp.float32(jnp.inf)
    pos_ref[...] = jnp.where(same & (row_ids != col_ids), dist, -inf)
    neg_ref[...] = jnp.where(same, inf, dist)


def _sc_reduce_body(pos_hbm, neg_hbm, pos_d_hbm, neg_d_hbm,
                    pos_i_hbm, neg_i_hbm,
                    pb0, nb0, pb1, nb1, spd, snd, spi, sni, sem0, sem1):
    c = lax.axis_index("c")
    s = lax.axis_index("s")
    wid = s * _NC + c
    base = wid * _ROWS_PER_W
    inf = jnp.float32(jnp.inf)
    big = jnp.float32(_N)
    lane = lax.broadcasted_iota(jnp.int32, (_L,), 0)
    lane_f = lane.astype(jnp.float32)

    pbufs, nbufs, sems = (pb0, pb1), (nb0, nb1), (sem0, sem1)

    def issue(r_local, b):
        return (pltpu.async_copy(pos_hbm.at[base + r_local], pbufs[b], sems[b]),
                pltpu.async_copy(neg_hbm.at[base + r_local], nbufs[b], sems[b]))

    handles = [None] * _ROWS_PER_W
    handles[0] = issue(0, 0)
    handles[1] = issue(1, 1)

    accs_out = []
    apd = api = and_ = ani = None
    for r_local in range(_ROWS_PER_W):
        t, k = divmod(r_local, _L)
        if k == 0:
            apd = jnp.zeros((_L,), jnp.float32)
            api = jnp.zeros((_L,), jnp.float32)
            and_ = jnp.zeros((_L,), jnp.float32)
            ani = jnp.zeros((_L,), jnp.float32)
        b = r_local % 2
        ha, hb = handles[r_local]
        ha.wait()
        hb.wait()
        pbuf, nbuf = pbufs[b], nbufs[b]

        def step(jj, accs, pbuf=pbuf, nbuf=nbuf):
            out = []
            for u in range(_UNROLL):
                bpv, bpi, bnv, bni = accs[u]
                ch = jj * _UNROLL + u
                colf = (ch * _L).astype(jnp.float32) + lane_f
                pv = pbuf[pl.ds(ch * _L, _L)]
                nv = nbuf[pl.ds(ch * _L, _L)]
                bpi = jnp.where(pv > bpv, colf, bpi)
                bpv = jnp.maximum(bpv, pv)
                bni = jnp.where(nv < bnv, colf, bni)
                bnv = jnp.minimum(bnv, nv)
                out.append((bpv, bpi, bnv, bni))
            return tuple(out)

        init1 = (jnp.full((_L,), -inf), jnp.zeros((_L,), jnp.float32),
                 jnp.full((_L,), inf), jnp.zeros((_L,), jnp.float32))
        accs = lax.fori_loop(0, _CHUNKS // _UNROLL, step, (init1,) * _UNROLL)

        if r_local + 2 < _ROWS_PER_W:
            handles[r_local + 2] = issue(r_local + 2, b)

        bv_p = accs[0][0]
        bv_n = accs[0][2]
        for u in range(1, _UNROLL):
            bv_p = jnp.maximum(bv_p, accs[u][0])
            bv_n = jnp.minimum(bv_n, accs[u][2])
        bi_p = jnp.full((_L,), big)
        bi_n = jnp.full((_L,), big)
        for u in range(_UNROLL):
            bi_p = jnp.minimum(bi_p, jnp.where(accs[u][0] == bv_p,
                                               accs[u][1], big))
            bi_n = jnp.minimum(bi_n, jnp.where(accs[u][2] == bv_n,
                                               accs[u][3], big))
        bi_p = jnp.where(bv_p == -inf, 0.0, bi_p)
        bi_n = jnp.where(bv_n == inf, 0.0, bi_n)

        pm = jnp.max(bv_p)
        pi = jnp.min(jnp.where(bv_p == pm, bi_p, big))
        nm = jnp.min(bv_n)
        ni = jnp.min(jnp.where(bv_n == nm, bi_n, big))

        in_lane = lane == k
        apd = jnp.where(in_lane, pm, apd)
        api = jnp.where(in_lane, pi, api)
        and_ = jnp.where(in_lane, nm, and_)
        ani = jnp.where(in_lane, ni, ani)

        if k == _L - 1:
            spd[pl.ds(t * _L, _L)] = apd
            spi[pl.ds(t * _L, _L)] = api
            snd[pl.ds(t * _L, _L)] = and_
            sni[pl.ds(t * _L, _L)] = ani

    pltpu.sync_copy(spd, pos_d_hbm.at[pl.ds(base, _ROWS_PER_W)])
    pltpu.sync_copy(snd, neg_d_hbm.at[pl.ds(base, _ROWS_PER_W)])
    pltpu.sync_copy(spi, pos_i_hbm.at[pl.ds(base, _ROWS_PER_W)])
    pltpu.sync_copy(sni, neg_i_hbm.at[pl.ds(base, _ROWS_PER_W)])


def kernel(embeddings, labels):
    n, d = embeddings.shape
    lab_col = labels.reshape(1, n)
    lab_row = labels.reshape(n, 1)

    pos_vals, neg_vals = pl.pallas_call(
        _vals_kernel,
        grid=(n // _BR,),
        in_specs=[
            pl.BlockSpec((n, d), lambda i: (0, 0)),
            pl.BlockSpec((_BR, 1), lambda i: (i, 0)),
            pl.BlockSpec((1, n), lambda i: (0, 0)),
        ],
        out_specs=(
            pl.BlockSpec((_BR, n), lambda i: (i, 0)),
            pl.BlockSpec((_BR, n), lambda i: (i, 0)),
        ),
        out_shape=(
            jax.ShapeDtypeStruct((n, n), jnp.float32),
            jax.ShapeDtypeStruct((n, n), jnp.float32),
        ),
        scratch_shapes=[pltpu.VMEM((n, d), jnp.float32)],
    )(embeddings, lab_row, lab_col)

    sc_reduce = functools.partial(
        pl.kernel,
        out_type=(
            jax.ShapeDtypeStruct((n,), jnp.float32),
            jax.ShapeDtypeStruct((n,), jnp.float32),
            jax.ShapeDtypeStruct((n,), jnp.float32),
            jax.ShapeDtypeStruct((n,), jnp.float32),
        ),
        mesh=plsc.VectorSubcoreMesh(core_axis_name="c", subcore_axis_name="s"),
        scratch_types=[
            pltpu.VMEM((_N,), jnp.float32),
            pltpu.VMEM((_N,), jnp.float32),
            pltpu.VMEM((_N,), jnp.float32),
            pltpu.VMEM((_N,), jnp.float32),
            pltpu.VMEM((_ROWS_PER_W,), jnp.float32),
            pltpu.VMEM((_ROWS_PER_W,), jnp.float32),
            pltpu.VMEM((_ROWS_PER_W,), jnp.float32),
            pltpu.VMEM((_ROWS_PER_W,), jnp.float32),
            pltpu.SemaphoreType.DMA,
            pltpu.SemaphoreType.DMA,
        ],
        compiler_params=pltpu.CompilerParams(needs_layout_passes=False),
    )(_sc_reduce_body)

    pos_d, neg_d, pos_if, neg_if = sc_reduce(pos_vals, neg_vals)

    anchors = jnp.arange(n, dtype=jnp.int32)
    triplets = jnp.column_stack((anchors, pos_if.astype(jnp.int32),
                                 neg_if.astype(jnp.int32)))
    return (triplets, pos_d, neg_d)


# speedup vs baseline: 1.4040x; 1.0347x over previous
import functools

import jax
import jax.numpy as jnp
from jax import lax
from jax.experimental import pallas as pl
from jax.experimental.pallas import tpu as pltpu
from jax.experimental.pallas import tpu_sc as plsc

_N = 1024
_NC, _NS, _L = 2, 16, 16
_NW = _NC * _NS
_ROWS_PER_W = _N // _NW
_GROUPS = _ROWS_PER_W // _L
_UNROLL = 8
_NBUF = 4
_CHUNKS = _N // _L
_BR = 256


def _vals_kernel(emb_full_ref, lab_row_ref, lab_col_ref,
                 pos_ref, neg_ref, en_ref):
    n = emb_full_ref.shape[0]
    i = pl.program_id(0)

    @pl.when(i == 0)
    def _():
        emb = emb_full_ref[...]
        sq = jnp.sum(emb * emb, axis=1, keepdims=True)
        en_ref[...] = emb * jax.lax.rsqrt(jnp.maximum(sq, 1e-30))

    en_full = en_ref[...]
    en_rows = en_ref[pl.ds(i * _BR, _BR), :]
    g = jax.lax.dot_general(en_rows, en_full, (((1,), (1,)), ((), ())),
                            preferred_element_type=jnp.float32,
                            precision=jax.lax.Precision.HIGHEST)
    dist = 1.0 - g

    lab_r = lab_row_ref[...]
    lab_c = lab_col_ref[...]
    row_ids = i * _BR + jax.lax.broadcasted_iota(jnp.int32, (_BR, n), 0)
    col_ids = jax.lax.broadcasted_iota(jnp.int32, (_BR, n), 1)
    same = lab_r == lab_c
    inf = jnp.float32(jnp.inf)
    pos_ref[...] = jnp.where(same & (row_ids != col_ids), dist, -inf)
    neg_ref[...] = jnp.where(same, inf, dist)


def _sc_reduce_body(pos_hbm, neg_hbm, pos_d_hbm, neg_d_hbm,
                    pos_i_hbm, neg_i_hbm,
                    pb0, nb0, pb1, nb1, pb2, nb2, pb3, nb3,
                    spd, snd, spi, sni, sem0, sem1, sem2, sem3):
    c = lax.axis_index("c")
    s = lax.axis_index("s")
    wid = s * _NC + c
    base = wid * _ROWS_PER_W
    inf = jnp.float32(jnp.inf)
    big = jnp.float32(_N)
    lane = lax.broadcasted_iota(jnp.int32, (_L,), 0)
    lane_f = lane.astype(jnp.float32)

    pbufs, nbufs = (pb0, pb1, pb2, pb3), (nb0, nb1, nb2, nb3)
    sems = (sem0, sem1, sem2, sem3)

    def issue(r_local, b):
        return (pltpu.async_copy(pos_hbm.at[base + r_local], pbufs[b], sems[b]),
                pltpu.async_copy(neg_hbm.at[base + r_local], nbufs[b], sems[b]))

    handles = [None] * _ROWS_PER_W
    for r0 in range(_NBUF):
        handles[r0] = issue(r0, r0)

    accs_out = []
    apd = api = and_ = ani = None
    for r_local in range(_ROWS_PER_W):
        t, k = divmod(r_local, _L)
        if k == 0:
            apd = jnp.zeros((_L,), jnp.float32)
            api = jnp.zeros((_L,), jnp.float32)
            and_ = jnp.zeros((_L,), jnp.float32)
            ani = jnp.zeros((_L,), jnp.float32)
        b = r_local % _NBUF
        ha, hb = handles[r_local]
        ha.wait()
        hb.wait()
        pbuf, nbuf = pbufs[b], nbufs[b]

        def step(jj, accs, pbuf=pbuf, nbuf=nbuf):
            out = []
            for u in range(_UNROLL):
                bpv, bpi, bnv, bni = accs[u]
                ch = jj * _UNROLL + u
                colf = (ch * _L).astype(jnp.float32) + lane_f
                pv = pbuf[pl.ds(ch * _L, _L)]
                nv = nbuf[pl.ds(ch * _L, _L)]
                bpi = jnp.where(pv > bpv, colf, bpi)
                bpv = jnp.maximum(bpv, pv)
                bni = jnp.where(nv < bnv, colf, bni)
                bnv = jnp.minimum(bnv, nv)
                out.append((bpv, bpi, bnv, bni))
            return tuple(out)

        init1 = (jnp.full((_L,), -inf), jnp.zeros((_L,), jnp.float32),
                 jnp.full((_L,), inf), jnp.zeros((_L,), jnp.float32))
        accs = lax.fori_loop(0, _CHUNKS // _UNROLL, step, (init1,) * _UNROLL)

        if r_local + _NBUF < _ROWS_PER_W:
            handles[r_local + _NBUF] = issue(r_local + _NBUF, b)

        bv_p = accs[0][0]
        bv_n = accs[0][2]
        for u in range(1, _UNROLL):
            bv_p = jnp.maximum(bv_p, accs[u][0])
            bv_n = jnp.minimum(bv_n, accs[u][2])
        bi_p = jnp.full((_L,), big)
        bi_n = jnp.full((_L,), big)
        for u in range(_UNROLL):
            bi_p = jnp.minimum(bi_p, jnp.where(accs[u][0] == bv_p,
                                               accs[u][1], big))
            bi_n = jnp.minimum(bi_n, jnp.where(accs[u][2] == bv_n,
                                               accs[u][3], big))
        bi_p = jnp.where(bv_p == -inf, 0.0, bi_p)
        bi_n = jnp.where(bv_n == inf, 0.0, bi_n)

        pm = jnp.max(bv_p)
        pi = jnp.min(jnp.where(bv_p == pm, bi_p, big))
        nm = jnp.min(bv_n)
        ni = jnp.min(jnp.where(bv_n == nm, bi_n, big))

        in_lane = lane == k
        apd = jnp.where(in_lane, pm, apd)
        api = jnp.where(in_lane, pi, api)
        and_ = jnp.where(in_lane, nm, and_)
        ani = jnp.where(in_lane, ni, ani)

        if k == _L - 1:
            spd[pl.ds(t * _L, _L)] = apd
            spi[pl.ds(t * _L, _L)] = api
            snd[pl.ds(t * _L, _L)] = and_
            sni[pl.ds(t * _L, _L)] = ani

    pltpu.sync_copy(spd, pos_d_hbm.at[pl.ds(base, _ROWS_PER_W)])
    pltpu.sync_copy(snd, neg_d_hbm.at[pl.ds(base, _ROWS_PER_W)])
    pltpu.sync_copy(spi, pos_i_hbm.at[pl.ds(base, _ROWS_PER_W)])
    pltpu.sync_copy(sni, neg_i_hbm.at[pl.ds(base, _ROWS_PER_W)])


def kernel(embeddings, labels):
    n, d = embeddings.shape
    lab_col = labels.reshape(1, n)
    lab_row = labels.reshape(n, 1)

    pos_vals, neg_vals = pl.pallas_call(
        _vals_kernel,
        grid=(n // _BR,),
        in_specs=[
            pl.BlockSpec((n, d), lambda i: (0, 0)),
            pl.BlockSpec((_BR, 1), lambda i: (i, 0)),
            pl.BlockSpec((1, n), lambda i: (0, 0)),
        ],
        out_specs=(
            pl.BlockSpec((_BR, n), lambda i: (i, 0)),
            pl.BlockSpec((_BR, n), lambda i: (i, 0)),
        ),
        out_shape=(
            jax.ShapeDtypeStruct((n, n), jnp.float32),
            jax.ShapeDtypeStruct((n, n), jnp.float32),
        ),
        scratch_shapes=[pltpu.VMEM((n, d), jnp.float32)],
    )(embeddings, lab_row, lab_col)

    sc_reduce = functools.partial(
        pl.kernel,
        out_type=(
            jax.ShapeDtypeStruct((n,), jnp.float32),
            jax.ShapeDtypeStruct((n,), jnp.float32),
            jax.ShapeDtypeStruct((n,), jnp.float32),
            jax.ShapeDtypeStruct((n,), jnp.float32),
        ),
        mesh=plsc.VectorSubcoreMesh(core_axis_name="c", subcore_axis_name="s"),
        scratch_types=(
            [pltpu.VMEM((_N,), jnp.float32) for _ in range(2 * _NBUF)]
            + [pltpu.VMEM((_ROWS_PER_W,), jnp.float32) for _ in range(4)]
            + [pltpu.SemaphoreType.DMA for _ in range(_NBUF)]
        ),
        compiler_params=pltpu.CompilerParams(needs_layout_passes=False),
    )(_sc_reduce_body)

    pos_d, neg_d, pos_if, neg_if = sc_reduce(pos_vals, neg_vals)

    anchors = jnp.arange(n, dtype=jnp.int32)
    triplets = jnp.column_stack((anchors, pos_if.astype(jnp.int32),
                                 neg_if.astype(jnp.int32)))
    return (triplets, pos_d, neg_d)


# speedup vs baseline: 1.6409x; 1.1688x over previous
import functools

import jax
import jax.numpy as jnp
from jax import lax
from jax.experimental import pallas as pl
from jax.experimental.pallas import tpu as pltpu
from jax.experimental.pallas import tpu_sc as plsc

_N = 1024
_NC, _NS, _L = 2, 16, 16
_NW = _NC * _NS
_SC_ROWS = 256
_SC_RPW = _SC_ROWS // _NW
_UNROLL = 8
_NBUF = 4
_CHUNKS = _N // _L
_BR = 256
_TC_BC = 256


def _vals_kernel(emb_full_ref, lab_row_ref, lab_col_ref,
                 pos_ref, neg_ref, en_ref):
    n = emb_full_ref.shape[0]
    i = pl.program_id(0)

    @pl.when(i == 0)
    def _():
        emb = emb_full_ref[...]
        sq = jnp.sum(emb * emb, axis=1, keepdims=True)
        en_ref[...] = emb * jax.lax.rsqrt(jnp.maximum(sq, 1e-30))

    en_full = en_ref[...]
    en_rows = en_ref[pl.ds(i * _BR, _BR), :]
    g = jax.lax.dot_general(en_rows, en_full, (((1,), (1,)), ((), ())),
                            preferred_element_type=jnp.float32,
                            precision=jax.lax.Precision.HIGHEST)
    dist = 1.0 - g

    lab_r = lab_row_ref[...]
    lab_c = lab_col_ref[...]
    row_ids = i * _BR + jax.lax.broadcasted_iota(jnp.int32, (_BR, n), 0)
    col_ids = jax.lax.broadcasted_iota(jnp.int32, (_BR, n), 1)
    same = lab_r == lab_c
    inf = jnp.float32(jnp.inf)
    pos_ref[...] = jnp.where(same & (row_ids != col_ids), dist, -inf)
    neg_ref[...] = jnp.where(same, inf, dist)


def _tc_reduce_kernel(pos_ref, neg_ref, pd_ref, nd_ref, pi_ref, ni_ref):
    n, bc = pos_ref.shape
    row_ids = jax.lax.broadcasted_iota(jnp.int32, (n, bc), 0)
    inf = jnp.float32(jnp.inf)

    pos = pos_ref[...]
    pm = jnp.max(pos, axis=0, keepdims=True)
    pi = jnp.min(jnp.where(pos == pm, row_ids, n), axis=0, keepdims=True)
    neg = neg_ref[...]
    nm = jnp.min(neg, axis=0, keepdims=True)
    ni = jnp.min(jnp.where(neg == nm, row_ids, n), axis=0, keepdims=True)

    pd_ref[...] = pm
    nd_ref[...] = nm
    pi_ref[...] = pi
    ni_ref[...] = ni


def _sc_reduce_body(pos_hbm, neg_hbm, pos_d_hbm, neg_d_hbm,
                    pos_i_hbm, neg_i_hbm,
                    pb0, nb0, pb1, nb1, pb2, nb2, pb3, nb3,
                    spd, snd, spi, sni, sem0, sem1, sem2, sem3):
    c = lax.axis_index("c")
    s = lax.axis_index("s")
    wid = s * _NC + c
    base = wid * _SC_RPW
    inf = jnp.float32(jnp.inf)
    big = jnp.float32(_N)
    lane = lax.broadcasted_iota(jnp.int32, (_L,), 0)
    lane_f = lane.astype(jnp.float32)

    pbufs, nbufs = (pb0, pb1, pb2, pb3), (nb0, nb1, nb2, nb3)
    sems = (sem0, sem1, sem2, sem3)

    def issue(r_local, b):
        return (pltpu.async_copy(pos_hbm.at[base + r_local], pbufs[b], sems[b]),
                pltpu.async_copy(neg_hbm.at[base + r_local], nbufs[b], sems[b]))

    handles = [None] * _SC_RPW
    for r0 in range(min(_NBUF, _SC_RPW)):
        handles[r0] = issue(r0, r0)

    apd = jnp.zeros((_L,), jnp.float32)
    api = jnp.zeros((_L,), jnp.float32)
    and_ = jnp.zeros((_L,), jnp.float32)
    ani = jnp.zeros((_L,), jnp.float32)
    for r_local in range(_SC_RPW):
        b = r_local % _NBUF
        ha, hb = handles[r_local]
        ha.wait()
        hb.wait()
        pbuf, nbuf = pbufs[b], nbufs[b]

        def step(jj, accs, pbuf=pbuf, nbuf=nbuf):
            out = []
            for u in range(_UNROLL):
                bpv, bpi, bnv, bni = accs[u]
                ch = jj * _UNROLL + u
                colf = (ch * _L).astype(jnp.float32) + lane_f
                pv = pbuf[pl.ds(ch * _L, _L)]
                nv = nbuf[pl.ds(ch * _L, _L)]
                bpi = jnp.where(pv > bpv, colf, bpi)
                bpv = jnp.maximum(bpv, pv)
                bni = jnp.where(nv < bnv, colf, bni)
                bnv = jnp.minimum(bnv, nv)
                out.append((bpv, bpi, bnv, bni))
            return tuple(out)

        init1 = (jnp.full((_L,), -inf), jnp.zeros((_L,), jnp.float32),
                 jnp.full((_L,), inf), jnp.zeros((_L,), jnp.float32))
        accs = lax.fori_loop(0, _CHUNKS // _UNROLL, step, (init1,) * _UNROLL)

        if r_local + _NBUF < _SC_RPW:
            handles[r_local + _NBUF] = issue(r_local + _NBUF, b)

        bv_p = accs[0][0]
        bv_n = accs[0][2]
        for u in range(1, _UNROLL):
            bv_p = jnp.maximum(bv_p, accs[u][0])
            bv_n = jnp.minimum(bv_n, accs[u][2])
        bi_p = jnp.full((_L,), big)
        bi_n = jnp.full((_L,), big)
        for u in range(_UNROLL):
            bi_p = jnp.minimum(bi_p, jnp.where(accs[u][0] == bv_p,
                                               accs[u][1], big))
            bi_n = jnp.minimum(bi_n, jnp.where(accs[u][2] == bv_n,
                                               accs[u][3], big))
        bi_p = jnp.where(bv_p == -inf, 0.0, bi_p)
        bi_n = jnp.where(bv_n == inf, 0.0, bi_n)

        pm = jnp.max(bv_p)
        pi = jnp.min(jnp.where(bv_p == pm, bi_p, big))
        nm = jnp.min(bv_n)
        ni = jnp.min(jnp.where(bv_n == nm, bi_n, big))

        in_lane = lane == r_local
        apd = jnp.where(in_lane, pm, apd)
        api = jnp.where(in_lane, pi, api)
        and_ = jnp.where(in_lane, nm, and_)
        ani = jnp.where(in_lane, ni, ani)

    spd[...] = apd
    spi[...] = api
    snd[...] = and_
    sni[...] = ani
    pltpu.sync_copy(spd, pos_d_hbm.at[wid])
    pltpu.sync_copy(snd, neg_d_hbm.at[wid])
    pltpu.sync_copy(spi, pos_i_hbm.at[wid])
    pltpu.sync_copy(sni, neg_i_hbm.at[wid])


def kernel(embeddings, labels):
    n, d = embeddings.shape
    lab_col = labels.reshape(1, n)
    lab_row = labels.reshape(n, 1)

    pos_vals, neg_vals = pl.pallas_call(
        _vals_kernel,
        grid=(n // _BR,),
        in_specs=[
            pl.BlockSpec((n, d), lambda i: (0, 0)),
            pl.BlockSpec((_BR, 1), lambda i: (i, 0)),
            pl.BlockSpec((1, n), lambda i: (0, 0)),
        ],
        out_specs=(
            pl.BlockSpec((_BR, n), lambda i: (i, 0)),
            pl.BlockSpec((_BR, n), lambda i: (i, 0)),
        ),
        out_shape=(
            jax.ShapeDtypeStruct((n, n), jnp.float32),
            jax.ShapeDtypeStruct((n, n), jnp.float32),
        ),
        scratch_shapes=[pltpu.VMEM((n, d), jnp.float32)],
    )(embeddings, lab_row, lab_col)

    sc_reduce = functools.partial(
        pl.kernel,
        out_type=tuple(jax.ShapeDtypeStruct((_NW, _L), jnp.float32)
                       for _ in range(4)),
        mesh=plsc.VectorSubcoreMesh(core_axis_name="c", subcore_axis_name="s"),
        scratch_types=(
            [pltpu.VMEM((_N,), jnp.float32) for _ in range(2 * _NBUF)]
            + [pltpu.VMEM((_L,), jnp.float32) for _ in range(4)]
            + [pltpu.SemaphoreType.DMA for _ in range(_NBUF)]
        ),
        compiler_params=pltpu.CompilerParams(needs_layout_passes=False),
    )(_sc_reduce_body)

    sc_pd, sc_nd, sc_pif, sc_nif = sc_reduce(pos_vals, neg_vals)

    ncols = n - _SC_ROWS
    tc_pd, tc_nd, tc_pi, tc_ni = pl.pallas_call(
        _tc_reduce_kernel,
        grid=(ncols // _TC_BC,),
        in_specs=[
            pl.BlockSpec((n, _TC_BC), lambda i: (0, i + _SC_ROWS // _TC_BC)),
            pl.BlockSpec((n, _TC_BC), lambda i: (0, i + _SC_ROWS // _TC_BC)),
        ],
        out_specs=tuple(pl.BlockSpec((1, _TC_BC), lambda i: (0, i))
                        for _ in range(4)),
        out_shape=(
            jax.ShapeDtypeStruct((1, ncols), jnp.float32),
            jax.ShapeDtypeStruct((1, ncols), jnp.float32),
            jax.ShapeDtypeStruct((1, ncols), jnp.int32),
            jax.ShapeDtypeStruct((1, ncols), jnp.int32),
        ),
    )(pos_vals, neg_vals)

    sc_take = _SC_RPW
    pos_d = jnp.concatenate([sc_pd[:, :sc_take].reshape(-1), tc_pd[0]])
    neg_d = jnp.concatenate([sc_nd[:, :sc_take].reshape(-1), tc_nd[0]])
    pos_i = jnp.concatenate([sc_pif[:, :sc_take].reshape(-1).astype(jnp.int32),
                             tc_pi[0]])
    neg_i = jnp.concatenate([sc_nif[:, :sc_take].reshape(-1).astype(jnp.int32),
                             tc_ni[0]])

    anchors = jnp.arange(n, dtype=jnp.int32)
    triplets = jnp.column_stack((anchors, pos_i, neg_i))
    return (triplets, pos_d, neg_d)


# speedup vs baseline: 1.9765x; 1.2045x over previous
import functools

import jax
import jax.numpy as jnp
from jax import lax
from jax.experimental import pallas as pl
from jax.experimental.pallas import tpu as pltpu
from jax.experimental.pallas import tpu_sc as plsc

_N = 1024
_NC, _NS, _L = 2, 16, 16
_NW = _NC * _NS
_SC_ROWS = 128
_SC_RPW = _SC_ROWS // _NW
_UNROLL = 8
_NBUF = 4
_CHUNKS = _N // _L


def _tc_kernel(emb_ref, lab_row_ref, lab_col_ref,
               pos_slab_ref, neg_slab_ref, pd_ref, nd_ref, pi_ref, ni_ref):
    n = emb_ref.shape[0]
    emb = emb_ref[...]
    sq = jnp.sum(emb * emb, axis=1, keepdims=True)
    en = emb * jax.lax.rsqrt(jnp.maximum(sq, 1e-30))
    g = jax.lax.dot_general(en, en, (((1,), (1,)), ((), ())),
                            preferred_element_type=jnp.float32,
                            precision=jax.lax.Precision.HIGHEST)
    dist = 1.0 - g

    lab_r = lab_row_ref[...]
    lab_c = lab_col_ref[...]
    row_ids = jax.lax.broadcasted_iota(jnp.int32, (n, n), 0)
    col_ids = jax.lax.broadcasted_iota(jnp.int32, (n, n), 1)
    same = lab_r == lab_c
    inf = jnp.float32(jnp.inf)
    pos_vals = jnp.where(same & (row_ids != col_ids), dist, -inf)
    neg_vals = jnp.where(same, inf, dist)

    pos_slab_ref[...] = pos_vals[:_SC_ROWS, :]
    neg_slab_ref[...] = neg_vals[:_SC_ROWS, :]

    pos_t = pos_vals[:, _SC_ROWS:]
    neg_t = neg_vals[:, _SC_ROWS:]
    rid_t = row_ids[:, _SC_ROWS:]
    pm = jnp.max(pos_t, axis=0, keepdims=True)
    pi = jnp.min(jnp.where(pos_t == pm, rid_t, n), axis=0, keepdims=True)
    nm = jnp.min(neg_t, axis=0, keepdims=True)
    ni = jnp.min(jnp.where(neg_t == nm, rid_t, n), axis=0, keepdims=True)
    pd_ref[...] = pm
    nd_ref[...] = nm
    pi_ref[...] = pi
    ni_ref[...] = ni


def _sc_reduce_body(pos_hbm, neg_hbm, pos_d_hbm, neg_d_hbm,
                    pos_i_hbm, neg_i_hbm,
                    pb0, nb0, pb1, nb1, pb2, nb2, pb3, nb3,
                    spd, snd, spi, sni, sem0, sem1, sem2, sem3):
    c = lax.axis_index("c")
    s = lax.axis_index("s")
    wid = s * _NC + c
    base = wid * _SC_RPW
    inf = jnp.float32(jnp.inf)
    big = jnp.float32(_N)
    lane = lax.broadcasted_iota(jnp.int32, (_L,), 0)
    lane_f = lane.astype(jnp.float32)

    pbufs, nbufs = (pb0, pb1, pb2, pb3), (nb0, nb1, nb2, nb3)
    sems = (sem0, sem1, sem2, sem3)

    def issue(r_local, b):
        return (pltpu.async_copy(pos_hbm.at[base + r_local], pbufs[b], sems[b]),
                pltpu.async_copy(neg_hbm.at[base + r_local], nbufs[b], sems[b]))

    handles = [None] * _SC_RPW
    for r0 in range(min(_NBUF, _SC_RPW)):
        handles[r0] = issue(r0, r0)

    apd = jnp.zeros((_L,), jnp.float32)
    api = jnp.zeros((_L,), jnp.float32)
    and_ = jnp.zeros((_L,), jnp.float32)
    ani = jnp.zeros((_L,), jnp.float32)
    for r_local in range(_SC_RPW):
        b = r_local % _NBUF
        ha, hb = handles[r_local]
        ha.wait()
        hb.wait()
        pbuf, nbuf = pbufs[b], nbufs[b]

        def step(jj, accs, pbuf=pbuf, nbuf=nbuf):
            out = []
            for u in range(_UNROLL):
                bpv, bpi, bnv, bni = accs[u]
                ch = jj * _UNROLL + u
                colf = (ch * _L).astype(jnp.float32) + lane_f
                pv = pbuf[pl.ds(ch * _L, _L)]
                nv = nbuf[pl.ds(ch * _L, _L)]
                bpi = jnp.where(pv > bpv, colf, bpi)
                bpv = jnp.maximum(bpv, pv)
                bni = jnp.where(nv < bnv, colf, bni)
                bnv = jnp.minimum(bnv, nv)
                out.append((bpv, bpi, bnv, bni))
            return tuple(out)

        init1 = (jnp.full((_L,), -inf), jnp.zeros((_L,), jnp.float32),
                 jnp.full((_L,), inf), jnp.zeros((_L,), jnp.float32))
        accs = lax.fori_loop(0, _CHUNKS // _UNROLL, step, (init1,) * _UNROLL)

        if r_local + _NBUF < _SC_RPW:
            handles[r_local + _NBUF] = issue(r_local + _NBUF, b)

        bv_p = accs[0][0]
        bv_n = accs[0][2]
        for u in range(1, _UNROLL):
            bv_p = jnp.maximum(bv_p, accs[u][0])
            bv_n = jnp.minimum(bv_n, accs[u][2])
        bi_p = jnp.full((_L,), big)
        bi_n = jnp.full((_L,), big)
        for u in range(_UNROLL):
            bi_p = jnp.minimum(bi_p, jnp.where(accs[u][0] == bv_p,
                                               accs[u][1], big))
            bi_n = jnp.minimum(bi_n, jnp.where(accs[u][2] == bv_n,
                                               accs[u][3], big))
        bi_p = jnp.where(bv_p == -inf, 0.0, bi_p)
        bi_n = jnp.where(bv_n == inf, 0.0, bi_n)

        pm = jnp.max(bv_p)
        pi = jnp.min(jnp.where(bv_p == pm, bi_p, big))
        nm = jnp.min(bv_n)
        ni = jnp.min(jnp.where(bv_n == nm, bi_n, big))

        in_lane = lane == r_local
        apd = jnp.where(in_lane, pm, apd)
        api = jnp.where(in_lane, pi, api)
        and_ = jnp.where(in_lane, nm, and_)
        ani = jnp.where(in_lane, ni, ani)

    spd[...] = apd
    spi[...] = api
    snd[...] = and_
    sni[...] = ani
    pltpu.sync_copy(spd, pos_d_hbm.at[wid])
    pltpu.sync_copy(snd, neg_d_hbm.at[wid])
    pltpu.sync_copy(spi, pos_i_hbm.at[wid])
    pltpu.sync_copy(sni, neg_i_hbm.at[wid])


def kernel(embeddings, labels):
    n, d = embeddings.shape
    lab_col = labels.reshape(1, n)
    lab_row = labels.reshape(n, 1)
    nt = n - _SC_ROWS

    pos_slab, neg_slab, tc_pd, tc_nd, tc_pi, tc_ni = pl.pallas_call(
        _tc_kernel,
        out_shape=(
            jax.ShapeDtypeStruct((_SC_ROWS, n), jnp.float32),
            jax.ShapeDtypeStruct((_SC_ROWS, n), jnp.float32),
            jax.ShapeDtypeStruct((1, nt), jnp.float32),
            jax.ShapeDtypeStruct((1, nt), jnp.float32),
            jax.ShapeDtypeStruct((1, nt), jnp.int32),
            jax.ShapeDtypeStruct((1, nt), jnp.int32),
        ),
    )(embeddings, lab_row, lab_col)

    sc_reduce = functools.partial(
        pl.kernel,
        out_type=tuple(jax.ShapeDtypeStruct((_NW, _L), jnp.float32)
                       for _ in range(4)),
        mesh=plsc.VectorSubcoreMesh(core_axis_name="c", subcore_axis_name="s"),
        scratch_types=(
            [pltpu.VMEM((_N,), jnp.float32) for _ in range(2 * _NBUF)]
            + [pltpu.VMEM((_L,), jnp.float32) for _ in range(4)]
            + [pltpu.SemaphoreType.DMA for _ in range(_NBUF)]
        ),
        compiler_params=pltpu.CompilerParams(needs_layout_passes=False),
    )(_sc_reduce_body)

    sc_pd, sc_nd, sc_pif, sc_nif = sc_reduce(pos_slab, neg_slab)

    k = _SC_RPW
    pos_d = jnp.concatenate([sc_pd[:, :k].reshape(-1), tc_pd[0]])
    neg_d = jnp.concatenate([sc_nd[:, :k].reshape(-1), tc_nd[0]])
    pos_i = jnp.concatenate([sc_pif[:, :k].reshape(-1).astype(jnp.int32),
                             tc_pi[0]])
    neg_i = jnp.concatenate([sc_nif[:, :k].reshape(-1).astype(jnp.int32),
                             tc_ni[0]])

    anchors = jnp.arange(n, dtype=jnp.int32)
    triplets = jnp.column_stack((anchors, pos_i, neg_i))
    return (triplets, pos_d, neg_d)


# speedup vs baseline: 1.9876x; 1.0056x over previous
import functools

import jax
import jax.numpy as jnp
from jax import lax
from jax.experimental import pallas as pl
from jax.experimental.pallas import tpu as pltpu
from jax.experimental.pallas import tpu_sc as plsc

_N = 1024
_NC, _NS, _L = 1, 16, 16
_NW = _NC * _NS
_SC_ROWS = 128
_SC_RPW = _SC_ROWS // _NW
_UNROLL = 8
_NBUF = 4
_CHUNKS = _N // _L


def _tc_kernel(emb_ref, lab_row_ref, lab_col_ref,
               pos_slab_ref, neg_slab_ref, pd_ref, nd_ref, pi_ref, ni_ref):
    n = emb_ref.shape[0]
    emb = emb_ref[...]
    sq = jnp.sum(emb * emb, axis=1, keepdims=True)
    en = emb * jax.lax.rsqrt(jnp.maximum(sq, 1e-30))
    g = jax.lax.dot_general(en, en, (((1,), (1,)), ((), ())),
                            preferred_element_type=jnp.float32,
                            precision=jax.lax.Precision.HIGHEST)
    dist = 1.0 - g

    lab_r = lab_row_ref[...]
    lab_c = lab_col_ref[...]
    row_ids = jax.lax.broadcasted_iota(jnp.int32, (n, n), 0)
    col_ids = jax.lax.broadcasted_iota(jnp.int32, (n, n), 1)
    same = lab_r == lab_c
    inf = jnp.float32(jnp.inf)
    pos_vals = jnp.where(same & (row_ids != col_ids), dist, -inf)
    neg_vals = jnp.where(same, inf, dist)

    pos_slab_ref[...] = pos_vals[:_SC_ROWS, :]
    neg_slab_ref[...] = neg_vals[:_SC_ROWS, :]

    pos_t = pos_vals[:, _SC_ROWS:]
    neg_t = neg_vals[:, _SC_ROWS:]
    rid_t = row_ids[:, _SC_ROWS:]
    pm = jnp.max(pos_t, axis=0, keepdims=True)
    pi = jnp.min(jnp.where(pos_t == pm, rid_t, n), axis=0, keepdims=True)
    nm = jnp.min(neg_t, axis=0, keepdims=True)
    ni = jnp.min(jnp.where(neg_t == nm, rid_t, n), axis=0, keepdims=True)
    pd_ref[...] = pm
    nd_ref[...] = nm
    pi_ref[...] = pi
    ni_ref[...] = ni


def _sc_reduce_body(pos_hbm, neg_hbm, pos_d_hbm, neg_d_hbm,
                    pos_i_hbm, neg_i_hbm,
                    pb0, nb0, pb1, nb1, pb2, nb2, pb3, nb3,
                    spd, snd, spi, sni, sem0, sem1, sem2, sem3):
    c = lax.axis_index("c")
    s = lax.axis_index("s")
    wid = s * _NC + c
    base = wid * _SC_RPW
    inf = jnp.float32(jnp.inf)
    big = jnp.float32(_N)
    lane = lax.broadcasted_iota(jnp.int32, (_L,), 0)
    lane_f = lane.astype(jnp.float32)

    pbufs, nbufs = (pb0, pb1, pb2, pb3), (nb0, nb1, nb2, nb3)
    sems = (sem0, sem1, sem2, sem3)

    def issue(r_local, b):
        return (pltpu.async_copy(pos_hbm.at[base + r_local], pbufs[b], sems[b]),
                pltpu.async_copy(neg_hbm.at[base + r_local], nbufs[b], sems[b]))

    handles = [None] * _SC_RPW
    for r0 in range(min(_NBUF, _SC_RPW)):
        handles[r0] = issue(r0, r0)

    apd = jnp.zeros((_L,), jnp.float32)
    api = jnp.zeros((_L,), jnp.float32)
    and_ = jnp.zeros((_L,), jnp.float32)
    ani = jnp.zeros((_L,), jnp.float32)
    for r_local in range(_SC_RPW):
        b = r_local % _NBUF
        ha, hb = handles[r_local]
        ha.wait()
        hb.wait()
        pbuf, nbuf = pbufs[b], nbufs[b]

        def step(jj, accs, pbuf=pbuf, nbuf=nbuf):
            out = []
            for u in range(_UNROLL):
                bpv, bpi, bnv, bni = accs[u]
                ch = jj * _UNROLL + u
                colf = (ch * _L).astype(jnp.float32) + lane_f
                pv = pbuf[pl.ds(ch * _L, _L)]
                nv = nbuf[pl.ds(ch * _L, _L)]
                bpi = jnp.where(pv > bpv, colf, bpi)
                bpv = jnp.maximum(bpv, pv)
                bni = jnp.where(nv < bnv, colf, bni)
                bnv = jnp.minimum(bnv, nv)
                out.append((bpv, bpi, bnv, bni))
            return tuple(out)

        init1 = (jnp.full((_L,), -inf), jnp.zeros((_L,), jnp.float32),
                 jnp.full((_L,), inf), jnp.zeros((_L,), jnp.float32))
        accs = lax.fori_loop(0, _CHUNKS // _UNROLL, step, (init1,) * _UNROLL)

        if r_local + _NBUF < _SC_RPW:
            handles[r_local + _NBUF] = issue(r_local + _NBUF, b)

        bv_p = accs[0][0]
        bv_n = accs[0][2]
        for u in range(1, _UNROLL):
            bv_p = jnp.maximum(bv_p, accs[u][0])
            bv_n = jnp.minimum(bv_n, accs[u][2])
        bi_p = jnp.full((_L,), big)
        bi_n = jnp.full((_L,), big)
        for u in range(_UNROLL):
            bi_p = jnp.minimum(bi_p, jnp.where(accs[u][0] == bv_p,
                                               accs[u][1], big))
            bi_n = jnp.minimum(bi_n, jnp.where(accs[u][2] == bv_n,
                                               accs[u][3], big))
        bi_p = jnp.where(bv_p == -inf, 0.0, bi_p)
        bi_n = jnp.where(bv_n == inf, 0.0, bi_n)

        pm = jnp.max(bv_p)
        pi = jnp.min(jnp.where(bv_p == pm, bi_p, big))
        nm = jnp.min(bv_n)
        ni = jnp.min(jnp.where(bv_n == nm, bi_n, big))

        in_lane = lane == r_local
        apd = jnp.where(in_lane, pm, apd)
        api = jnp.where(in_lane, pi, api)
        and_ = jnp.where(in_lane, nm, and_)
        ani = jnp.where(in_lane, ni, ani)

    spd[...] = apd
    spi[...] = api
    snd[...] = and_
    sni[...] = ani
    pltpu.sync_copy(spd, pos_d_hbm.at[wid])
    pltpu.sync_copy(snd, neg_d_hbm.at[wid])
    pltpu.sync_copy(spi, pos_i_hbm.at[wid])
    pltpu.sync_copy(sni, neg_i_hbm.at[wid])


def kernel(embeddings, labels):
    n, d = embeddings.shape
    lab_col = labels.reshape(1, n)
    lab_row = labels.reshape(n, 1)
    nt = n - _SC_ROWS

    pos_slab, neg_slab, tc_pd, tc_nd, tc_pi, tc_ni = pl.pallas_call(
        _tc_kernel,
        out_shape=(
            jax.ShapeDtypeStruct((_SC_ROWS, n), jnp.float32),
            jax.ShapeDtypeStruct((_SC_ROWS, n), jnp.float32),
            jax.ShapeDtypeStruct((1, nt), jnp.float32),
            jax.ShapeDtypeStruct((1, nt), jnp.float32),
            jax.ShapeDtypeStruct((1, nt), jnp.int32),
            jax.ShapeDtypeStruct((1, nt), jnp.int32),
        ),
    )(embeddings, lab_row, lab_col)

    sc_reduce = functools.partial(
        pl.kernel,
        out_type=tuple(jax.ShapeDtypeStruct((_NW, _L), jnp.float32)
                       for _ in range(4)),
        mesh=plsc.VectorSubcoreMesh(core_axis_name="c", subcore_axis_name="s", num_cores=1),
        scratch_types=(
            [pltpu.VMEM((_N,), jnp.float32) for _ in range(2 * _NBUF)]
            + [pltpu.VMEM((_L,), jnp.float32) for _ in range(4)]
            + [pltpu.SemaphoreType.DMA for _ in range(_NBUF)]
        ),
        compiler_params=pltpu.CompilerParams(needs_layout_passes=False),
    )(_sc_reduce_body)

    sc_pd, sc_nd, sc_pif, sc_nif = sc_reduce(pos_slab, neg_slab)

    k = _SC_RPW
    pos_d = jnp.concatenate([sc_pd[:, :k].reshape(-1), tc_pd[0]])
    neg_d = jnp.concatenate([sc_nd[:, :k].reshape(-1), tc_nd[0]])
    pos_i = jnp.concatenate([sc_pif[:, :k].reshape(-1).astype(jnp.int32),
                             tc_pi[0]])
    neg_i = jnp.concatenate([sc_nif[:, :k].reshape(-1).astype(jnp.int32),
                             tc_ni[0]])

    anchors = jnp.arange(n, dtype=jnp.int32)
    triplets = jnp.column_stack((anchors, pos_i, neg_i))
    return (triplets, pos_d, neg_d)
